# baseline (speedup 1.0000x reference)
import functools

import jax
import jax.numpy as jnp
from jax import lax
from jax.experimental import pallas as pl
from jax.experimental.pallas import tpu as pltpu
from jax.experimental.pallas import tpu_sc as plsc

D = 128
NC, NS = 2, 16
L = 16
NW = NC * NS
B = 128
_BIG = 2 ** 30

_MESH = plsc.VectorSubcoreMesh(
    core_axis_name="c", subcore_axis_name="s", num_cores=NC, num_subcores=NS)


def _relu(x):
    return jnp.maximum(x, 0.0)


def _fill_zero(zref):
    zv = jnp.zeros((L,), jnp.float32)

    def body(i, _):
        for j in range(D // L):
            zref[i, pl.ds(j * L, L)] = zv
        return 0

    lax.fori_loop(0, zref.shape[0], body, 0)


def _pair_gather(na, nbt, src, dst):
    E = src.shape[0]
    per_w = E // NW
    gb = 80
    nb = per_w // gb
    assert per_w % gb == 0 and E % NW == 0

    @functools.partial(
        pl.kernel,
        out_type=jax.ShapeDtypeStruct((E, D), jnp.float32),
        mesh=_MESH,
        scratch_types=[
            pltpu.VMEM((gb,), jnp.int32),
            pltpu.VMEM((gb,), jnp.int32),
            pltpu.VMEM((gb, D), jnp.float32),
            pltpu.VMEM((gb, D), jnp.float32),
            pltpu.SemaphoreType.DMA,
            pltpu.SemaphoreType.DMA,
        ],
    )
    def k(na_hbm, nb_hbm, src_hbm, dst_hbm, out_hbm, si, di, ra, rb, s1, s2):
        c = lax.axis_index("c")
        s = lax.axis_index("s")
        w = s * NC + c
        base_w = w * per_w

        def body(i, _):
            base = base_w + i * gb
            pltpu.sync_copy(src_hbm.at[pl.ds(base, gb)], si)
            pltpu.sync_copy(dst_hbm.at[pl.ds(base, gb)], di)
            cpa = pltpu.async_copy(na_hbm.at[si], ra, s1)
            cpb = pltpu.async_copy(nb_hbm.at[di], rb, s2)
            cpa.wait()
            cpb.wait()

            def add_row(r, _):
                for j in range(D // L):
                    ra[r, pl.ds(j * L, L)] = (
                        ra[r, pl.ds(j * L, L)] + rb[r, pl.ds(j * L, L)])
                return 0

            lax.fori_loop(0, gb, add_row, 0)
            pltpu.sync_copy(ra, out_hbm.at[pl.ds(base, gb)])
            return 0

        lax.fori_loop(0, nb, body, 0)

    return k(na, nbt, src, dst)


def _segsum(table, gid, sid, S, CH, wb_tiles, wb_rows):
    M_pad = gid.shape[0]
    assert M_pad % (NS * L) == 0
    slab = M_pad // NS
    n_chunks = S // CH
    assert n_chunks * CH == S and n_chunks % NC == 0
    ncps = n_chunks // NC
    assert wb_tiles * wb_rows == CH
    cap = slab + 2 * B

    @functools.partial(
        pl.kernel,
        out_type=jax.ShapeDtypeStruct((S, D), jnp.float32),
        mesh=_MESH,
        scratch_types=[
            pltpu.VMEM((slab,), jnp.int32),
            pltpu.VMEM((slab,), jnp.int32),
            pltpu.VMEM((cap,), jnp.int32),
            pltpu.VMEM((cap,), jnp.int32),
            pltpu.VMEM((B,), jnp.int32),
            pltpu.VMEM((B,), jnp.int32),
            pltpu.VMEM((B, D), jnp.float32),
            pltpu.VMEM((B, D), jnp.float32),
            pltpu.VMEM_SHARED((CH + 8, D), jnp.float32),
            pltpu.SemaphoreType.DMA,
        ],
        compiler_params=pltpu.CompilerParams(needs_layout_passes=False),
    )
    def k(gid_hbm, sid_hbm, table_hbm, out_hbm,
          gid_v, sid_v, selg, sels, idxg, idxs, rows, zero, acc, sem):
        c = lax.axis_index("c")
        s = lax.axis_index("s")
        pltpu.sync_copy(gid_hbm.at[pl.ds(s * slab, slab)], gid_v)
        pltpu.sync_copy(sid_hbm.at[pl.ds(s * slab, slab)], sid_v)
        _fill_zero(zero)
        nvec = slab // L
        iot = lax.iota(jnp.int32, L)
        zi = jnp.zeros((L,), jnp.int32)
        di = jnp.full((L,), CH, jnp.int32)

        def chunk_body(ci, _):
            lo = (c * ncps + ci) * CH
            hi = lo + CH

            @pl.when(s < wb_tiles)
            def _():
                zb = s * wb_rows
                off = 0
                rem = wb_rows
                while rem > 0:
                    blk = min(rem, B)
                    pltpu.sync_copy(zero.at[pl.ds(0, blk)],
                                    acc.at[pl.ds(zb + off, blk)])
                    off += blk
                    rem -= blk

            plsc.subcore_barrier()

            def scan_body(i, cnt):
                g = gid_v[pl.ds(i * L, L)]
                sv = sid_v[pl.ds(i * L, L)]
                m = (sv >= lo) & (sv < hi)
                mi = m.astype(jnp.int32)
                pos = plsc.cumsum(mi)
                idx = cnt + pos - 1
                plsc.store_scatter(selg, [idx], g, mask=m)
                plsc.store_scatter(sels, [idx], sv - lo, mask=m)
                return cnt + jnp.max(pos)

            cnt = lax.fori_loop(0, nvec, scan_body, jnp.int32(0))

            for j in range(B // L):
                idxv = cnt + j * L + iot
                plsc.store_scatter(selg, [idxv], zi)
                plsc.store_scatter(sels, [idxv], di)

            def drain_body(b, _):
                off = b * B
                for j in range(B // L):
                    idxs[pl.ds(j * L, L)] = sels[pl.ds(off + j * L, L)]
                pltpu.async_copy(
                    table_hbm.at[selg.at[pl.ds(off, B)]], rows, sem).wait()
                pltpu.sync_copy(rows, acc.at[idxs], add=True)
                return 0

            lax.fori_loop(0, (cnt + B - 1) // B, drain_body, 0)
            plsc.subcore_barrier()

            @pl.when(s < wb_tiles)
            def _():
                wb = s * wb_rows
                pltpu.sync_copy(acc.at[pl.ds(wb, wb_rows)],
                                out_hbm.at[pl.ds(lo + wb, wb_rows)])

            plsc.subcore_barrier()
            return 0

        lax.fori_loop(0, ncps, chunk_body, 0)

    return k(gid, sid, table)


def _dot(a, b):
    return jnp.dot(a, b, preferred_element_type=jnp.float32)


def _row_spec(blk):
    return pl.BlockSpec((blk, D), lambda i: (i, 0))


def _full_spec(shape):
    return pl.BlockSpec(shape, lambda i: (0,) * len(shape))


def _tc_tables(node_rep, wa, wb, be):
    n = node_rep.shape[0]
    blk = 1000

    def body(x_ref, wa_ref, wb_ref, be_ref, na_ref, nb_ref):
        x = x_ref[...]
        na_ref[...] = _dot(x, wa_ref[...])
        nb_ref[...] = _dot(x, wb_ref[...]) + be_ref[...]

    return pl.pallas_call(
        body,
        grid=(n // blk,),
        in_specs=[_row_spec(blk), _full_spec((D, D)), _full_spec((D, D)),
                  _full_spec((1, D))],
        out_specs=[_row_spec(blk), _row_spec(blk)],
        out_shape=[jax.ShapeDtypeStruct((n, D), jnp.float32),
                   jax.ShapeDtypeStruct((n, D), jnp.float32)],
    )(node_rep, wa, wb, be)


def _tc_node(node_rep, agg, wna, wnb, bn):
    n = node_rep.shape[0]
    blk = 1000

    def body(x_ref, a_ref, wa_ref, wb_ref, b_ref, o_ref):
        x = x_ref[...]
        h = _dot(x, wa_ref[...]) + _dot(a_ref[...], wb_ref[...]) + b_ref[...]
        o_ref[...] = x + _relu(h)

    return pl.pallas_call(
        body,
        grid=(n // blk,),
        in_specs=[_row_spec(blk), _row_spec(blk), _full_spec((D, D)),
                  _full_spec((D, D)), _full_spec((1, D))],
        out_specs=_row_spec(blk),
        out_shape=jax.ShapeDtypeStruct((n, D), jnp.float32),
    )(node_rep, agg, wna, wnb, bn)


def _tc_cycle(cycle_rep, e2c, wc1, bc1, wc2, bc2, scale_row):
    n = cycle_rep.shape[0]
    blk = 1000

    def body(x_ref, a_ref, w1_ref, b1_ref, w2_ref, b2_ref, sc_ref, o_ref):
        x = x_ref[...]
        h = x * sc_ref[...] + a_ref[...]
        t = _relu(_dot(h, w1_ref[...]) + b1_ref[...])
        o_ref[...] = x + _relu(_dot(t, w2_ref[...]) + b2_ref[...])

    return pl.pallas_call(
        body,
        grid=(n // blk,),
        in_specs=[_row_spec(blk), _row_spec(blk), _full_spec((D, D)),
                  _full_spec((1, D)), _full_spec((D, D)), _full_spec((1, D)),
                  _full_spec((1, D))],
        out_specs=_row_spec(blk),
        out_shape=jax.ShapeDtypeStruct((n, D), jnp.float32),
    )(cycle_rep, e2c, wc1, bc1, wc2, bc2, scale_row)


def _tc_edge(pre, edge_rep, c2e, wec1, bec1, wec2, bec2,
             m1a, m1b, bm1, wm2, bm2, scale_row):
    n = edge_rep.shape[0]
    blk = 512

    def body(p_ref, e_ref, a_ref, w1_ref, b1_ref, w2_ref, b2_ref,
             ma_ref, mb_ref, bm_ref, wm_ref, bo_ref, sc_ref, o_ref):
        e = e_ref[...]
        x1 = _relu(p_ref[...])
        he = e * sc_ref[...] + a_ref[...]
        t = _relu(_dot(he, w1_ref[...]) + b1_ref[...])
        e2 = _relu(_dot(t, w2_ref[...]) + b2_ref[...])
        u = _relu(_dot(x1, ma_ref[...]) + _dot(e2, mb_ref[...]) + bm_ref[...])
        o_ref[...] = e + _dot(u, wm_ref[...]) + bo_ref[...]

    return pl.pallas_call(
        body,
        grid=(n // blk,),
        in_specs=[_row_spec(blk), _row_spec(blk), _row_spec(blk),
                  _full_spec((D, D)), _full_spec((1, D)),
                  _full_spec((D, D)), _full_spec((1, D)),
                  _full_spec((D, D)), _full_spec((D, D)), _full_spec((1, D)),
                  _full_spec((D, D)), _full_spec((1, D)), _full_spec((1, D))],
        out_specs=_row_spec(blk),
        out_shape=jax.ShapeDtypeStruct((n, D), jnp.float32),
    )(pre, edge_rep, c2e, wec1, bec1, wec2, bec2, m1a, m1b, bm1, wm2, bm2,
      scale_row)


def kernel(node_rep, edge_rep, cycle_rep, edge_index, cycle_ids,
           cycle_edge_ids, W_e1, b_e1, W_n, b_n, W_ec1, b_ec1, W_ec2, b_ec2,
           eps_e, W_c1, b_c1, W_c2, b_c2, eps_c, W_m1, b_m1, W_m2, b_m2):
    E = edge_rep.shape[0]
    N = node_rep.shape[0]
    C = cycle_rep.shape[0]
    M = cycle_ids.shape[0]

    src = edge_index[0].astype(jnp.int32)
    dst = edge_index[1].astype(jnp.int32)
    cids = cycle_ids.astype(jnp.int32)
    ceids = cycle_edge_ids.astype(jnp.int32)

    grp = NS * L
    m_pad = -(-M // grp) * grp
    pad = m_pad - M
    zpad = jnp.zeros((pad,), jnp.int32)
    bpad = jnp.full((pad,), _BIG, jnp.int32)
    cids_g = jnp.concatenate([cids, zpad])
    cids_s = jnp.concatenate([cids, bpad])
    ceids_g = jnp.concatenate([ceids, zpad])
    ceids_s = jnp.concatenate([ceids, bpad])
    eid = jnp.arange(E, dtype=jnp.int32)

    na, nbt = _tc_tables(node_rep, W_e1[:D], W_e1[D:], b_e1[None])

    pre = _pair_gather(na, nbt, src, dst)
    node_agg = _segsum(edge_rep, eid, dst, S=N, CH=1000,
                       wb_tiles=5, wb_rows=200)
    c2e = _segsum(cycle_rep, cids_g, ceids_s, S=E, CH=8000,
                  wb_tiles=8, wb_rows=1000)
    e2c = _segsum(edge_rep, ceids_g, cids_s, S=C, CH=2000,
                  wb_tiles=5, wb_rows=400)

    scale_e = (1.0 + eps_e) * jnp.ones((1, D), jnp.float32)
    scale_c = (1.0 + eps_c) * jnp.ones((1, D), jnp.float32)
    node_out = _tc_node(node_rep, node_agg, W_n[:D], W_n[D:], b_n[None])
    cycle_out = _tc_cycle(cycle_rep, e2c, W_c1, b_c1[None], W_c2, b_c2[None],
                          scale_c)
    edge_out = _tc_edge(pre, edge_rep, c2e, W_ec1, b_ec1[None], W_ec2,
                        b_ec2[None], W_m1[:D], W_m1[D:], b_m1[None], W_m2,
                        b_m2[None], scale_e)
    return (node_out, edge_out, cycle_out)


# speedup vs baseline: 1.1670x; 1.1670x over previous
import functools

import jax
import jax.numpy as jnp
from jax import lax
from jax.experimental import pallas as pl
from jax.experimental.pallas import tpu as pltpu
from jax.experimental.pallas import tpu_sc as plsc

D = 128
NC, NS = 2, 16
L = 16
NW = NC * NS
B = 128
_BIG = 2 ** 30

_MESH = plsc.VectorSubcoreMesh(
    core_axis_name="c", subcore_axis_name="s", num_cores=NC, num_subcores=NS)


def _relu(x):
    return jnp.maximum(x, 0.0)


def _fill_zero(zref):
    zv = jnp.zeros((L,), jnp.float32)

    def body(i, _):
        for j in range(D // L):
            zref[i, pl.ds(j * L, L)] = zv
        return 0

    lax.fori_loop(0, zref.shape[0], body, 0)


def _pair_gather(na, nbt, src, dst):
    E = src.shape[0]
    per_w = E // NW
    gb = 80
    nb = per_w // gb
    assert per_w % gb == 0 and E % NW == 0

    @functools.partial(
        pl.kernel,
        out_type=jax.ShapeDtypeStruct((E, D), jnp.float32),
        mesh=_MESH,
        scratch_types=[
            pltpu.VMEM((per_w,), jnp.int32),
            pltpu.VMEM((per_w,), jnp.int32),
            pltpu.VMEM((gb, D), jnp.float32),
            pltpu.VMEM((gb, D), jnp.float32),
            pltpu.SemaphoreType.DMA,
            pltpu.SemaphoreType.DMA,
        ],
    )
    def k(na_hbm, nb_hbm, src_hbm, dst_hbm, out_hbm, si, di, ra, rb, s1, s2):
        c = lax.axis_index("c")
        s = lax.axis_index("s")
        w = s * NC + c
        base_w = w * per_w
        pltpu.sync_copy(src_hbm.at[pl.ds(base_w, per_w)], si)
        pltpu.sync_copy(dst_hbm.at[pl.ds(base_w, per_w)], di)

        def body(i, _):
            off = i * gb
            cpa = pltpu.async_copy(na_hbm.at[si.at[pl.ds(off, gb)]], ra, s1)
            cpb = pltpu.async_copy(nb_hbm.at[di.at[pl.ds(off, gb)]], rb, s2)
            cpa.wait()
            cpb.wait()

            @plsc.parallel_loop(0, gb, unroll=2)
            def _(r):
                for j in range(D // L):
                    ra[r, pl.ds(j * L, L)] = (
                        ra[r, pl.ds(j * L, L)] + rb[r, pl.ds(j * L, L)])

            pltpu.sync_copy(ra, out_hbm.at[pl.ds(base_w + off, gb)])
            return 0

        lax.fori_loop(0, nb, body, 0)

    return k(na, nbt, src, dst)


def _rowsum_partial(table, sid2d, S, wb_tiles, wb_rows):
    R = sid2d.shape[0]
    rows_sc = R // NC
    rows_t = rows_sc // NS
    assert rows_t * NS * NC == R
    assert wb_tiles * wb_rows == S

    @functools.partial(
        pl.kernel,
        out_type=jax.ShapeDtypeStruct((NC * S, D), jnp.float32),
        mesh=_MESH,
        scratch_types=[
            pltpu.VMEM((rows_t, B), jnp.int32),
            pltpu.VMEM((B, D), jnp.float32),
            pltpu.VMEM((B, D), jnp.float32),
            pltpu.SemaphoreType.DMA,
            pltpu.SemaphoreType.DMA,
            pltpu.SemaphoreType.DMA,
            pltpu.SemaphoreType.DMA,
            pltpu.VMEM_SHARED((S + 8, D), jnp.float32),
        ],
        compiler_params=pltpu.CompilerParams(needs_layout_passes=False),
    )
    def k(sid_hbm, table_hbm, out_hbm, sidv, rows0, rows1, sg0, sg1, ss0, ss1,
          acc):
        c = lax.axis_index("c")
        s = lax.axis_index("s")
        r0 = (c * NS + s) * rows_t
        pltpu.sync_copy(sid_hbm.at[pl.ds(r0, rows_t)], sidv)
        _fill_zero(rows0)

        @pl.when(s < wb_tiles)
        def _():
            zb = s * wb_rows
            off = 0
            rem = wb_rows
            while rem > 0:
                blk = min(rem, B)
                pltpu.sync_copy(rows0.at[pl.ds(0, blk)],
                                acc.at[pl.ds(zb + off, blk)])
                off += blk
                rem -= blk

        plsc.subcore_barrier()

        tmax = table.shape[0] - B

        def pair_body(p, _):
            e0 = jnp.minimum((r0 + 2 * p) * B, tmax)
            e1 = jnp.minimum((r0 + 2 * p + 1) * B, tmax)
            cp0 = pltpu.async_copy(table_hbm.at[pl.ds(e0, B)], rows0, sg0)
            cp1 = pltpu.async_copy(table_hbm.at[pl.ds(e1, B)], rows1, sg1)
            cp0.wait()
            a0 = pltpu.async_copy(rows0, acc.at[sidv.at[2 * p]], ss0,
                                  add=True)
            cp1.wait()
            a1 = pltpu.async_copy(rows1, acc.at[sidv.at[2 * p + 1]], ss1,
                                  add=True)
            a0.wait()
            a1.wait()
            return 0

        lax.fori_loop(0, rows_t // 2, pair_body, 0)
        plsc.subcore_barrier()

        @pl.when(s < wb_tiles)
        def _():
            wb = s * wb_rows
            pltpu.sync_copy(acc.at[pl.ds(wb, wb_rows)],
                            out_hbm.at[pl.ds(c * S + wb, wb_rows)])

    return k(sid2d, table)


def _segsum(table, gid, sid, S, CH, wb_tiles, wb_rows, bs=64):
    M_pad = sid.shape[0]
    assert M_pad % (NS * L) == 0
    slab = M_pad // NS
    n_chunks = S // CH
    assert n_chunks * CH == S and n_chunks % NC == 0
    ncps = n_chunks // NC
    assert wb_tiles * wb_rows == CH
    cap = slab + 2 * bs
    cap_r = -(-cap // bs)
    has_gid = gid is not None

    scratch = [
        pltpu.VMEM((slab,), jnp.int32),
        pltpu.VMEM((cap,), jnp.int32),
        pltpu.VMEM((cap_r, bs), jnp.int32),
        pltpu.VMEM((bs, D), jnp.float32),
        pltpu.VMEM((bs, D), jnp.float32),
        pltpu.SemaphoreType.DMA,
        pltpu.SemaphoreType.DMA,
        pltpu.SemaphoreType.DMA,
        pltpu.SemaphoreType.DMA,
    ]
    if has_gid:
        scratch.insert(0, pltpu.VMEM((slab,), jnp.int32))
    multi = ncps > 1
    if multi:
        scratch.append(pltpu.VMEM((bs, D), jnp.float32))
    scratch.append(pltpu.VMEM_SHARED((CH + 8, D), jnp.float32))

    def k(*refs):
        if has_gid:
            (gid_hbm, sid_hbm, table_hbm, out_hbm, gid_v, sid_v, selg, sels,
             rows0, rows1, sg0, sg1, ss0, ss1, *zero_acc) = refs
        else:
            (sid_hbm, table_hbm, out_hbm, sid_v, selg, sels,
             rows0, rows1, sg0, sg1, ss0, ss1, *zero_acc) = refs
        if multi:
            zero, acc = zero_acc
        else:
            acc, = zero_acc
            zero = rows0
        c = lax.axis_index("c")
        s = lax.axis_index("s")
        if has_gid:
            pltpu.sync_copy(gid_hbm.at[pl.ds(s * slab, slab)], gid_v)
        pltpu.sync_copy(sid_hbm.at[pl.ds(s * slab, slab)], sid_v)
        _fill_zero(zero)
        nvec = slab // L
        iot = lax.iota(jnp.int32, L)
        zi = jnp.zeros((L,), jnp.int32)
        dvi = jnp.full((L,), CH, jnp.int32)
        sh = bs.bit_length() - 1
        gbase = s * slab

        def chunk_body(ci, _):
            lo = (c * ncps + ci) * CH
            hi = lo + CH

            @pl.when(s < wb_tiles)
            def _():
                zb = s * wb_rows
                off = 0
                rem = wb_rows
                while rem > 0:
                    blk = min(rem, bs)
                    pltpu.sync_copy(zero.at[pl.ds(0, blk)],
                                    acc.at[pl.ds(zb + off, blk)])
                    off += blk
                    rem -= blk

            plsc.subcore_barrier()

            @plsc.parallel_loop(0, nvec, unroll=2,
                                carry=jnp.zeros((L,), jnp.int32))
            def cnt_vec(i, cnt):
                sv = sid_v[pl.ds(i * L, L)]
                if has_gid:
                    g = gid_v[pl.ds(i * L, L)]
                else:
                    g = gbase + i * L + iot
                m = (sv >= lo) & (sv < hi)
                pos = plsc.cumsum(m.astype(jnp.int32))
                idx = cnt + pos - 1
                plsc.store_scatter(selg, [idx], g, mask=m)
                plsc.store_scatter(sels, [idx >> sh, idx & (bs - 1)],
                                   sv - lo, mask=m)
                return cnt + plsc.all_reduce_population_count(m)

            cnt = jnp.max(cnt_vec)

            for j in range(2 * bs // L):
                idxv = cnt + j * L + iot
                plsc.store_scatter(selg, [idxv], zi)
                plsc.store_scatter(sels, [idxv >> sh, idxv & (bs - 1)], dvi)

            def drain_body(p, _):
                o0 = p * (2 * bs)
                o1 = o0 + bs
                cp0 = pltpu.async_copy(
                    table_hbm.at[selg.at[pl.ds(o0, bs)]], rows0, sg0)
                cp1 = pltpu.async_copy(
                    table_hbm.at[selg.at[pl.ds(o1, bs)]], rows1, sg1)
                cp0.wait()
                a0 = pltpu.async_copy(rows0, acc.at[sels.at[2 * p]], ss0,
                                      add=True)
                cp1.wait()
                a1 = pltpu.async_copy(rows1, acc.at[sels.at[2 * p + 1]], ss1,
                                      add=True)
                a0.wait()
                a1.wait()
                return 0

            lax.fori_loop(0, (cnt + 2 * bs - 1) // (2 * bs), drain_body, 0)
            plsc.subcore_barrier()

            @pl.when(s < wb_tiles)
            def _():
                wb = s * wb_rows
                pltpu.sync_copy(acc.at[pl.ds(wb, wb_rows)],
                                out_hbm.at[pl.ds(lo + wb, wb_rows)])

            plsc.subcore_barrier()
            return 0

        lax.fori_loop(0, ncps, chunk_body, 0)

    kk = functools.partial(
        pl.kernel,
        out_type=jax.ShapeDtypeStruct((S, D), jnp.float32),
        mesh=_MESH,
        scratch_types=scratch,
        compiler_params=pltpu.CompilerParams(needs_layout_passes=False),
    )(k)
    if has_gid:
        return kk(gid, sid, table)
    return kk(sid, table)


def _dot(a, b):
    return jnp.dot(a, b, preferred_element_type=jnp.float32)


def _row_spec(blk):
    return pl.BlockSpec((blk, D), lambda i: (i, 0))


def _full_spec(shape):
    return pl.BlockSpec(shape, lambda i: (0,) * len(shape))


def _tc_tables(node_rep, wa, wb, be):
    n = node_rep.shape[0]
    blk = 1000

    def body(x_ref, wa_ref, wb_ref, be_ref, na_ref, nb_ref):
        x = x_ref[...]
        na_ref[...] = _dot(x, wa_ref[...])
        nb_ref[...] = _dot(x, wb_ref[...]) + be_ref[...]

    return pl.pallas_call(
        body,
        grid=(n // blk,),
        in_specs=[_row_spec(blk), _full_spec((D, D)), _full_spec((D, D)),
                  _full_spec((1, D))],
        out_specs=[_row_spec(blk), _row_spec(blk)],
        out_shape=[jax.ShapeDtypeStruct((n, D), jnp.float32),
                   jax.ShapeDtypeStruct((n, D), jnp.float32)],
    )(node_rep, wa, wb, be)


def _tc_node(node_rep, parts, wna, wnb, bn):
    n = node_rep.shape[0]
    blk = 1000
    nblk = n // blk

    def body(x_ref, a_ref, a2_ref, wa_ref, wb_ref, b_ref, o_ref):
        x = x_ref[...]
        agg = a_ref[...] + a2_ref[...]
        h = _dot(x, wa_ref[...]) + _dot(agg, wb_ref[...]) + b_ref[...]
        o_ref[...] = x + _relu(h)

    return pl.pallas_call(
        body,
        grid=(nblk,),
        in_specs=[_row_spec(blk),
                  pl.BlockSpec((blk, D), lambda i: (i, 0)),
                  pl.BlockSpec((blk, D), lambda i: (i + nblk, 0)),
                  _full_spec((D, D)), _full_spec((D, D)), _full_spec((1, D))],
        out_specs=_row_spec(blk),
        out_shape=jax.ShapeDtypeStruct((n, D), jnp.float32),
    )(node_rep, parts, parts, wna, wnb, bn)


def _tc_cycle(cycle_rep, e2c, wc1, bc1, wc2, bc2, scale_row):
    n = cycle_rep.shape[0]
    blk = 1000

    def body(x_ref, a_ref, w1_ref, b1_ref, w2_ref, b2_ref, sc_ref, o_ref):
        x = x_ref[...]
        h = x * sc_ref[...] + a_ref[...]
        t = _relu(_dot(h, w1_ref[...]) + b1_ref[...])
        o_ref[...] = x + _relu(_dot(t, w2_ref[...]) + b2_ref[...])

    return pl.pallas_call(
        body,
        grid=(n // blk,),
        in_specs=[_row_spec(blk), _row_spec(blk), _full_spec((D, D)),
                  _full_spec((1, D)), _full_spec((D, D)), _full_spec((1, D)),
                  _full_spec((1, D))],
        out_specs=_row_spec(blk),
        out_shape=jax.ShapeDtypeStruct((n, D), jnp.float32),
    )(cycle_rep, e2c, wc1, bc1, wc2, bc2, scale_row)


def _tc_edge(pre, edge_rep, c2e, wec1, bec1, wec2, bec2,
             m1a, m1b, bm1, wm2, bm2, scale_row):
    n = edge_rep.shape[0]
    blk = 512

    def body(p_ref, e_ref, a_ref, w1_ref, b1_ref, w2_ref, b2_ref,
             ma_ref, mb_ref, bm_ref, wm_ref, bo_ref, sc_ref, o_ref):
        e = e_ref[...]
        x1 = _relu(p_ref[...])
        he = e * sc_ref[...] + a_ref[...]
        t = _relu(_dot(he, w1_ref[...]) + b1_ref[...])
        e2 = _relu(_dot(t, w2_ref[...]) + b2_ref[...])
        u = _relu(_dot(x1, ma_ref[...]) + _dot(e2, mb_ref[...]) + bm_ref[...])
        o_ref[...] = e + _dot(u, wm_ref[...]) + bo_ref[...]

    return pl.pallas_call(
        body,
        grid=(n // blk,),
        in_specs=[_row_spec(blk), _row_spec(blk), _row_spec(blk),
                  _full_spec((D, D)), _full_spec((1, D)),
                  _full_spec((D, D)), _full_spec((1, D)),
                  _full_spec((D, D)), _full_spec((D, D)), _full_spec((1, D)),
                  _full_spec((D, D)), _full_spec((1, D)), _full_spec((1, D))],
        out_specs=_row_spec(blk),
        out_shape=jax.ShapeDtypeStruct((n, D), jnp.float32),
    )(pre, edge_rep, c2e, wec1, bec1, wec2, bec2, m1a, m1b, bm1, wm2, bm2,
      scale_row)


def kernel(node_rep, edge_rep, cycle_rep, edge_index, cycle_ids,
           cycle_edge_ids, W_e1, b_e1, W_n, b_n, W_ec1, b_ec1, W_ec2, b_ec2,
           eps_e, W_c1, b_c1, W_c2, b_c2, eps_c, W_m1, b_m1, W_m2, b_m2):
    E = edge_rep.shape[0]
    N = node_rep.shape[0]
    C = cycle_rep.shape[0]
    M = cycle_ids.shape[0]

    src = edge_index[0].astype(jnp.int32)
    dst = edge_index[1].astype(jnp.int32)
    cids = cycle_ids.astype(jnp.int32)
    ceids = cycle_edge_ids.astype(jnp.int32)

    grp = NS * L
    m_pad = -(-M // grp) * grp
    pad = m_pad - M
    zpad = jnp.zeros((pad,), jnp.int32)
    bpad = jnp.full((pad,), _BIG, jnp.int32)
    cids_g = jnp.concatenate([cids, zpad])
    cids_s = jnp.concatenate([cids, bpad])
    ceids_g = jnp.concatenate([ceids, zpad])
    ceids_s = jnp.concatenate([ceids, bpad])
    eid = jnp.arange(E, dtype=jnp.int32)

    na, nbt = _tc_tables(node_rep, W_e1[:D], W_e1[D:], b_e1[None])

    pre = _pair_gather(na, nbt, src, dst)
    assert E % B == 0
    e_rows = -(-(E // B) // (NW * 8)) * NW * 8
    dst_pad = jnp.concatenate(
        [dst, jnp.full((e_rows * B - E,), N, jnp.int32)]).reshape(e_rows, B)
    node_parts = _rowsum_partial(edge_rep, dst_pad, S=N,
                                 wb_tiles=10, wb_rows=1000)
    c2e = _segsum(cycle_rep, cids_g, ceids_s, S=E, CH=8000,
                  wb_tiles=8, wb_rows=1000)
    e2c = _segsum(edge_rep, ceids_g, cids_s, S=C, CH=2000,
                  wb_tiles=10, wb_rows=200)

    scale_e = (1.0 + eps_e) * jnp.ones((1, D), jnp.float32)
    scale_c = (1.0 + eps_c) * jnp.ones((1, D), jnp.float32)
    node_out = _tc_node(node_rep, node_parts, W_n[:D], W_n[D:], b_n[None])
    cycle_out = _tc_cycle(cycle_rep, e2c, W_c1, b_c1[None], W_c2, b_c2[None],
                          scale_c)
    edge_out = _tc_edge(pre, edge_rep, c2e, W_ec1, b_ec1[None], W_ec2,
                        b_ec2[None], W_m1[:D], W_m1[D:], b_m1[None], W_m2,
                        b_m2[None], scale_e)
    return (node_out, edge_out, cycle_out)


# speedup vs baseline: 1.1681x; 1.0009x over previous
import functools

import jax
import jax.numpy as jnp
from jax import lax
from jax.experimental import pallas as pl
from jax.experimental.pallas import tpu as pltpu
from jax.experimental.pallas import tpu_sc as plsc

D = 128
NC, NS = 2, 16
L = 16
NW = NC * NS
B = 128
_BIG = 2 ** 30

_MESH = plsc.VectorSubcoreMesh(
    core_axis_name="c", subcore_axis_name="s", num_cores=NC, num_subcores=NS)


def _relu(x):
    return jnp.maximum(x, 0.0)


def _fill_zero(zref):
    zv = jnp.zeros((L,), jnp.float32)

    def body(i, _):
        for j in range(D // L):
            zref[i, pl.ds(j * L, L)] = zv
        return 0

    lax.fori_loop(0, zref.shape[0], body, 0)


def _pair_gather(na, nbt, src, dst):
    E = src.shape[0]
    per_w = E // NW
    gb = 80
    nb = per_w // gb
    assert per_w % gb == 0 and E % NW == 0

    @functools.partial(
        pl.kernel,
        out_type=jax.ShapeDtypeStruct((E, D), jnp.float32),
        mesh=_MESH,
        scratch_types=[
            pltpu.VMEM((per_w,), jnp.int32),
            pltpu.VMEM((per_w,), jnp.int32),
            pltpu.VMEM((gb, D), jnp.float32),
            pltpu.VMEM((gb, D), jnp.float32),
            pltpu.SemaphoreType.DMA,
            pltpu.SemaphoreType.DMA,
        ],
    )
    def k(na_hbm, nb_hbm, src_hbm, dst_hbm, out_hbm, si, di, ra, rb, s1, s2):
        c = lax.axis_index("c")
        s = lax.axis_index("s")
        w = s * NC + c
        base_w = w * per_w
        pltpu.sync_copy(src_hbm.at[pl.ds(base_w, per_w)], si)
        pltpu.sync_copy(dst_hbm.at[pl.ds(base_w, per_w)], di)

        def body(i, _):
            off = i * gb
            cpa = pltpu.async_copy(na_hbm.at[si.at[pl.ds(off, gb)]], ra, s1)
            cpb = pltpu.async_copy(nb_hbm.at[di.at[pl.ds(off, gb)]], rb, s2)
            cpa.wait()
            cpb.wait()

            @plsc.parallel_loop(0, gb, unroll=2)
            def _(r):
                for j in range(D // L):
                    ra[r, pl.ds(j * L, L)] = (
                        ra[r, pl.ds(j * L, L)] + rb[r, pl.ds(j * L, L)])

            pltpu.sync_copy(ra, out_hbm.at[pl.ds(base_w + off, gb)])
            return 0

        lax.fori_loop(0, nb, body, 0)

    return k(na, nbt, src, dst)


def _rowsum_partial(table, sid2d, S, wb_tiles, wb_rows):
    R = sid2d.shape[0]
    rows_sc = R // NC
    rows_t = rows_sc // NS
    assert rows_t * NS * NC == R
    assert wb_tiles * wb_rows == S

    @functools.partial(
        pl.kernel,
        out_type=jax.ShapeDtypeStruct((NC * S, D), jnp.float32),
        mesh=_MESH,
        scratch_types=[
            pltpu.VMEM((rows_t, B), jnp.int32),
            pltpu.VMEM((B, D), jnp.float32),
            pltpu.VMEM((B, D), jnp.float32),
            pltpu.SemaphoreType.DMA,
            pltpu.SemaphoreType.DMA,
            pltpu.SemaphoreType.DMA,
            pltpu.SemaphoreType.DMA,
            pltpu.VMEM_SHARED((S + 8, D), jnp.float32),
        ],
        compiler_params=pltpu.CompilerParams(needs_layout_passes=False),
    )
    def k(sid_hbm, table_hbm, out_hbm, sidv, rows0, rows1, sg0, sg1, ss0, ss1,
          acc):
        c = lax.axis_index("c")
        s = lax.axis_index("s")
        r0 = (c * NS + s) * rows_t
        pltpu.sync_copy(sid_hbm.at[pl.ds(r0, rows_t)], sidv)
        _fill_zero(rows0)

        @pl.when(s < wb_tiles)
        def _():
            zb = s * wb_rows
            off = 0
            rem = wb_rows
            while rem > 0:
                blk = min(rem, B)
                pltpu.sync_copy(rows0.at[pl.ds(0, blk)],
                                acc.at[pl.ds(zb + off, blk)])
                off += blk
                rem -= blk

        plsc.subcore_barrier()

        tmax = table.shape[0] - B

        def pair_body(p, _):
            e0 = jnp.minimum((r0 + 2 * p) * B, tmax)
            e1 = jnp.minimum((r0 + 2 * p + 1) * B, tmax)
            cp0 = pltpu.async_copy(table_hbm.at[pl.ds(e0, B)], rows0, sg0)
            cp1 = pltpu.async_copy(table_hbm.at[pl.ds(e1, B)], rows1, sg1)
            cp0.wait()
            a0 = pltpu.async_copy(rows0, acc.at[sidv.at[2 * p]], ss0,
                                  add=True)
            cp1.wait()
            a1 = pltpu.async_copy(rows1, acc.at[sidv.at[2 * p + 1]], ss1,
                                  add=True)
            a0.wait()
            a1.wait()
            return 0

        lax.fori_loop(0, rows_t // 2, pair_body, 0)
        plsc.subcore_barrier()

        @pl.when(s < wb_tiles)
        def _():
            wb = s * wb_rows
            pltpu.sync_copy(acc.at[pl.ds(wb, wb_rows)],
                            out_hbm.at[pl.ds(c * S + wb, wb_rows)])

    return k(sid2d, table)


def _segsum(table, gid, sid, S, CH, wb_tiles, wb_rows, bs=64):
    M_pad = sid.shape[0]
    assert M_pad % (NS * L) == 0
    slab = M_pad // NS
    n_chunks = S // CH
    assert n_chunks * CH == S and n_chunks % NC == 0
    ncps = n_chunks // NC
    assert wb_tiles * wb_rows == CH
    cap = slab + 2 * bs
    cap_r = -(-cap // bs)
    has_gid = gid is not None

    scratch = [
        pltpu.VMEM((slab,), jnp.int32),
        pltpu.VMEM((cap,), jnp.int32),
        pltpu.VMEM((cap_r, bs), jnp.int32),
        pltpu.VMEM((bs, D), jnp.float32),
        pltpu.VMEM((bs, D), jnp.float32),
        pltpu.SemaphoreType.DMA,
        pltpu.SemaphoreType.DMA,
        pltpu.SemaphoreType.DMA,
        pltpu.SemaphoreType.DMA,
    ]
    if has_gid:
        scratch.insert(0, pltpu.VMEM((slab,), jnp.int32))
    multi = ncps > 1
    scratch.append(pltpu.VMEM_SHARED((CH + 8, D), jnp.float32))

    def k(*refs):
        if has_gid:
            (gid_hbm, sid_hbm, table_hbm, out_hbm, gid_v, sid_v, selg, sels,
             rows0, rows1, sg0, sg1, ss0, ss1, acc) = refs
        else:
            (sid_hbm, table_hbm, out_hbm, sid_v, selg, sels,
             rows0, rows1, sg0, sg1, ss0, ss1, acc) = refs
        zero = rows0
        c = lax.axis_index("c")
        s = lax.axis_index("s")
        if has_gid:
            pltpu.sync_copy(gid_hbm.at[pl.ds(s * slab, slab)], gid_v)
        pltpu.sync_copy(sid_hbm.at[pl.ds(s * slab, slab)], sid_v)
        _fill_zero(zero)
        nvec = slab // L
        iot = lax.iota(jnp.int32, L)
        zi = jnp.zeros((L,), jnp.int32)
        dvi = jnp.full((L,), CH, jnp.int32)
        sh = bs.bit_length() - 1
        gbase = s * slab

        def chunk_body(ci, _):
            lo = (c * ncps + ci) * CH
            hi = lo + CH
            if multi:
                _fill_zero(zero)

            @pl.when(s < wb_tiles)
            def _():
                zb = s * wb_rows
                off = 0
                rem = wb_rows
                while rem > 0:
                    blk = min(rem, bs)
                    pltpu.sync_copy(zero.at[pl.ds(0, blk)],
                                    acc.at[pl.ds(zb + off, blk)])
                    off += blk
                    rem -= blk

            plsc.subcore_barrier()

            @plsc.parallel_loop(0, nvec, unroll=4,
                                carry=jnp.zeros((L,), jnp.int32))
            def cnt_vec(i, cnt):
                sv = sid_v[pl.ds(i * L, L)]
                if has_gid:
                    g = gid_v[pl.ds(i * L, L)]
                else:
                    g = gbase + i * L + iot
                m = (sv >= lo) & (sv < hi)
                pos = plsc.cumsum(m.astype(jnp.int32))
                idx = cnt + pos - 1
                plsc.store_scatter(selg, [idx], g, mask=m)
                plsc.store_scatter(sels, [idx >> sh, idx & (bs - 1)],
                                   sv - lo, mask=m)
                return cnt + plsc.all_reduce_population_count(m)

            cnt = jnp.max(cnt_vec)

            for j in range(2 * bs // L):
                idxv = cnt + j * L + iot
                plsc.store_scatter(selg, [idxv], zi)
                plsc.store_scatter(sels, [idxv >> sh, idxv & (bs - 1)], dvi)

            def drain_body(p, _):
                o0 = p * (2 * bs)
                o1 = o0 + bs
                cp0 = pltpu.async_copy(
                    table_hbm.at[selg.at[pl.ds(o0, bs)]], rows0, sg0)
                cp1 = pltpu.async_copy(
                    table_hbm.at[selg.at[pl.ds(o1, bs)]], rows1, sg1)
                cp0.wait()
                a0 = pltpu.async_copy(rows0, acc.at[sels.at[2 * p]], ss0,
                                      add=True)
                cp1.wait()
                a1 = pltpu.async_copy(rows1, acc.at[sels.at[2 * p + 1]], ss1,
                                      add=True)
                a0.wait()
                a1.wait()
                return 0

            lax.fori_loop(0, (cnt + 2 * bs - 1) // (2 * bs), drain_body, 0)
            plsc.subcore_barrier()

            @pl.when(s < wb_tiles)
            def _():
                wb = s * wb_rows
                pltpu.sync_copy(acc.at[pl.ds(wb, wb_rows)],
                                out_hbm.at[pl.ds(lo + wb, wb_rows)])

            plsc.subcore_barrier()
            return 0

        lax.fori_loop(0, ncps, chunk_body, 0)

    kk = functools.partial(
        pl.kernel,
        out_type=jax.ShapeDtypeStruct((S, D), jnp.float32),
        mesh=_MESH,
        scratch_types=scratch,
        compiler_params=pltpu.CompilerParams(needs_layout_passes=False),
    )(k)
    if has_gid:
        return kk(gid, sid, table)
    return kk(sid, table)


def _dot(a, b):
    return jnp.dot(a, b, preferred_element_type=jnp.float32)


def _row_spec(blk):
    return pl.BlockSpec((blk, D), lambda i: (i, 0))


def _full_spec(shape):
    return pl.BlockSpec(shape, lambda i: (0,) * len(shape))


def _tc_tables(node_rep, wa, wb, be):
    n = node_rep.shape[0]
    blk = 1000

    def body(x_ref, wa_ref, wb_ref, be_ref, na_ref, nb_ref):
        x = x_ref[...]
        na_ref[...] = _dot(x, wa_ref[...])
        nb_ref[...] = _dot(x, wb_ref[...]) + be_ref[...]

    return pl.pallas_call(
        body,
        grid=(n // blk,),
        in_specs=[_row_spec(blk), _full_spec((D, D)), _full_spec((D, D)),
                  _full_spec((1, D))],
        out_specs=[_row_spec(blk), _row_spec(blk)],
        out_shape=[jax.ShapeDtypeStruct((n, D), jnp.float32),
                   jax.ShapeDtypeStruct((n, D), jnp.float32)],
    )(node_rep, wa, wb, be)


def _tc_node(node_rep, parts, wna, wnb, bn):
    n = node_rep.shape[0]
    blk = 1000
    nblk = n // blk

    def body(x_ref, a_ref, a2_ref, wa_ref, wb_ref, b_ref, o_ref):
        x = x_ref[...]
        agg = a_ref[...] + a2_ref[...]
        h = _dot(x, wa_ref[...]) + _dot(agg, wb_ref[...]) + b_ref[...]
        o_ref[...] = x + _relu(h)

    return pl.pallas_call(
        body,
        grid=(nblk,),
        in_specs=[_row_spec(blk),
                  pl.BlockSpec((blk, D), lambda i: (i, 0)),
                  pl.BlockSpec((blk, D), lambda i: (i + nblk, 0)),
                  _full_spec((D, D)), _full_spec((D, D)), _full_spec((1, D))],
        out_specs=_row_spec(blk),
        out_shape=jax.ShapeDtypeStruct((n, D), jnp.float32),
    )(node_rep, parts, parts, wna, wnb, bn)


def _tc_cycle(cycle_rep, e2c, wc1, bc1, wc2, bc2, scale_row):
    n = cycle_rep.shape[0]
    blk = 1000

    def body(x_ref, a_ref, w1_ref, b1_ref, w2_ref, b2_ref, sc_ref, o_ref):
        x = x_ref[...]
        h = x * sc_ref[...] + a_ref[...]
        t = _relu(_dot(h, w1_ref[...]) + b1_ref[...])
        o_ref[...] = x + _relu(_dot(t, w2_ref[...]) + b2_ref[...])

    return pl.pallas_call(
        body,
        grid=(n // blk,),
        in_specs=[_row_spec(blk), _row_spec(blk), _full_spec((D, D)),
                  _full_spec((1, D)), _full_spec((D, D)), _full_spec((1, D)),
                  _full_spec((1, D))],
        out_specs=_row_spec(blk),
        out_shape=jax.ShapeDtypeStruct((n, D), jnp.float32),
    )(cycle_rep, e2c, wc1, bc1, wc2, bc2, scale_row)


def _tc_edge(pre, edge_rep, c2e, wec1, bec1, wec2, bec2,
             m1a, m1b, bm1, wm2, bm2, scale_row):
    n = edge_rep.shape[0]
    blk = 512

    def body(p_ref, e_ref, a_ref, w1_ref, b1_ref, w2_ref, b2_ref,
             ma_ref, mb_ref, bm_ref, wm_ref, bo_ref, sc_ref, o_ref):
        e = e_ref[...]
        x1 = _relu(p_ref[...])
        he = e * sc_ref[...] + a_ref[...]
        t = _relu(_dot(he, w1_ref[...]) + b1_ref[...])
        e2 = _relu(_dot(t, w2_ref[...]) + b2_ref[...])
        u = _relu(_dot(x1, ma_ref[...]) + _dot(e2, mb_ref[...]) + bm_ref[...])
        o_ref[...] = e + _dot(u, wm_ref[...]) + bo_ref[...]

    return pl.pallas_call(
        body,
        grid=(n // blk,),
        in_specs=[_row_spec(blk), _row_spec(blk), _row_spec(blk),
                  _full_spec((D, D)), _full_spec((1, D)),
                  _full_spec((D, D)), _full_spec((1, D)),
                  _full_spec((D, D)), _full_spec((D, D)), _full_spec((1, D)),
                  _full_spec((D, D)), _full_spec((1, D)), _full_spec((1, D))],
        out_specs=_row_spec(blk),
        out_shape=jax.ShapeDtypeStruct((n, D), jnp.float32),
    )(pre, edge_rep, c2e, wec1, bec1, wec2, bec2, m1a, m1b, bm1, wm2, bm2,
      scale_row)


def kernel(node_rep, edge_rep, cycle_rep, edge_index, cycle_ids,
           cycle_edge_ids, W_e1, b_e1, W_n, b_n, W_ec1, b_ec1, W_ec2, b_ec2,
           eps_e, W_c1, b_c1, W_c2, b_c2, eps_c, W_m1, b_m1, W_m2, b_m2):
    E = edge_rep.shape[0]
    N = node_rep.shape[0]
    C = cycle_rep.shape[0]
    M = cycle_ids.shape[0]

    src = edge_index[0].astype(jnp.int32)
    dst = edge_index[1].astype(jnp.int32)
    cids = cycle_ids.astype(jnp.int32)
    ceids = cycle_edge_ids.astype(jnp.int32)

    grp = NS * L * 4
    m_pad = -(-M // grp) * grp
    pad = m_pad - M
    zpad = jnp.zeros((pad,), jnp.int32)
    bpad = jnp.full((pad,), _BIG, jnp.int32)
    cids_g = jnp.concatenate([cids, zpad])
    cids_s = jnp.concatenate([cids, bpad])
    ceids_g = jnp.concatenate([ceids, zpad])
    ceids_s = jnp.concatenate([ceids, bpad])
    eid = jnp.arange(E, dtype=jnp.int32)

    na, nbt = _tc_tables(node_rep, W_e1[:D], W_e1[D:], b_e1[None])

    pre = _pair_gather(na, nbt, src, dst)
    assert E % B == 0
    e_rows = -(-(E // B) // (NW * 8)) * NW * 8
    dst_pad = jnp.concatenate(
        [dst, jnp.full((e_rows * B - E,), N, jnp.int32)]).reshape(e_rows, B)
    node_parts = _rowsum_partial(edge_rep, dst_pad, S=N,
                                 wb_tiles=10, wb_rows=1000)
    c2e = _segsum(cycle_rep, cids_g, ceids_s, S=E, CH=8000,
                  wb_tiles=8, wb_rows=1000)
    e2c = _segsum(edge_rep, ceids_g, cids_s, S=C, CH=2000,
                  wb_tiles=10, wb_rows=200)

    scale_e = (1.0 + eps_e) * jnp.ones((1, D), jnp.float32)
    scale_c = (1.0 + eps_c) * jnp.ones((1, D), jnp.float32)
    node_out = _tc_node(node_rep, node_parts, W_n[:D], W_n[D:], b_n[None])
    cycle_out = _tc_cycle(cycle_rep, e2c, W_c1, b_c1[None], W_c2, b_c2[None],
                          scale_c)
    edge_out = _tc_edge(pre, edge_rep, c2e, W_ec1, b_ec1[None], W_ec2,
                        b_ec2[None], W_m1[:D], W_m1[D:], b_m1[None], W_m2,
                        b_m2[None], scale_e)
    return (node_out, edge_out, cycle_out)
